# baseline (speedup 1.0000x reference)
import functools

import jax
import jax.numpy as jnp
from jax import lax
from jax.experimental import pallas as pl
from jax.experimental.pallas import tpu as pltpu
from jax.experimental.pallas import tpu_sc as plsc

N = 10000
E = 320000
D = 128
H = 128
G = 128

_NSC = 2
_NTILE = 16
_KCH = 128
NP = N + 112


def _mm_body(x_ref, w_ref, b_ref, o_ref, *, relu):
    acc = jnp.dot(x_ref[...], w_ref[...], preferred_element_type=jnp.float32)
    acc = acc + b_ref[...]
    o_ref[...] = jnp.maximum(acc, 0.0) if relu else acc


def mm_bias(x, w, b, relu=False):
    m = x.shape[0]
    blk = 1000 if m % 1000 == 0 else m
    return pl.pallas_call(
        functools.partial(_mm_body, relu=relu),
        grid=(m // blk,),
        in_specs=[
            pl.BlockSpec((blk, H), lambda i: (i, 0)),
            pl.BlockSpec((H, H), lambda i: (0, 0)),
            pl.BlockSpec((1, H), lambda i: (0, 0)),
        ],
        out_specs=pl.BlockSpec((blk, H), lambda i: (i, 0)),
        out_shape=jax.ShapeDtypeStruct((m, H), jnp.float32),
    )(x, w, b.reshape(1, H))


_NBUF = 4
_EALIGN = _NSC * _NTILE * _KCH * _NBUF
_SLICE = _NSC * _NTILE * _KCH
_EPAD_MAX = -(-E // _EALIGN) * _EALIGN


def _make_prop(n_acc, e_pad):
    ept = e_pad // (_NSC * _NTILE)
    iters = ept // _KCH
    nsup = iters // _NBUF
    rpt = n_acc // _NTILE
    mesh = plsc.VectorSubcoreMesh(core_axis_name="c", subcore_axis_name="s")
    scratch = (
        [pltpu.VMEM((_KCH,), jnp.int32) for _ in range(2 * _NBUF)]
        + [pltpu.VMEM((_KCH, H), jnp.float32) for _ in range(_NBUF)]
        + [pltpu.VMEM_SHARED((n_acc, H), jnp.float32)]
        + [pltpu.SemaphoreType.DMA for _ in range(3 * _NBUF)]
    )

    @functools.partial(
        pl.kernel,
        mesh=mesh,
        out_type=jax.ShapeDtypeStruct((_NSC, n_acc, H), jnp.float32),
        scratch_types=scratch,
    )
    def prop(feat, srcp, dstp, zrows, out, *refs):
        src_v = refs[0:_NBUF]
        dst_v = refs[_NBUF:2 * _NBUF]
        rows_v = refs[2 * _NBUF:3 * _NBUF]
        acc = refs[3 * _NBUF]
        sem_i = refs[3 * _NBUF + 1: 4 * _NBUF + 1]
        sem_g = refs[4 * _NBUF + 1: 5 * _NBUF + 1]
        sem_s = refs[5 * _NBUF + 1: 6 * _NBUF + 1]
        c = lax.axis_index("c")
        s = lax.axis_index("s")
        pltpu.sync_copy(zrows, acc.at[pl.ds(s * rpt, rpt)])
        plsc.subcore_barrier()
        base = (c * _NTILE + s) * ept

        def idx_start(chunk, b):
            off = base + chunk * _KCH
            pltpu.async_copy(srcp.at[pl.ds(off, _KCH)], src_v[b], sem_i[b])
            pltpu.async_copy(dstp.at[pl.ds(off, _KCH)], dst_v[b], sem_i[b])

        for b in range(_NBUF):
            idx_start(b, b)

        def body(g, carry):
            for b in range(_NBUF):
                pltpu.make_async_copy(
                    srcp.at[pl.ds(0, _KCH)], src_v[b], sem_i[b]).wait()
                pltpu.make_async_copy(
                    dstp.at[pl.ds(0, _KCH)], dst_v[b], sem_i[b]).wait()
                pltpu.async_copy(feat.at[src_v[b]], rows_v[b], sem_g[b])
            for b in range(_NBUF):
                pltpu.make_async_copy(
                    feat.at[src_v[b]], rows_v[b], sem_g[b]).wait()
                pltpu.async_copy(rows_v[b], acc.at[dst_v[b]], sem_s[b],
                                 add=True)
            for b in range(_NBUF):
                pltpu.make_async_copy(
                    rows_v[b], acc.at[dst_v[b]], sem_s[b]).wait()

                @pl.when(g + 1 < nsup)
                def _():
                    idx_start((g + 1) * _NBUF + b, b)
            return carry

        lax.fori_loop(0, nsup, body, 0)
        plsc.subcore_barrier()
        pltpu.sync_copy(acc.at[pl.ds(s * rpt, rpt)],
                        out.at[c, pl.ds(s * rpt, rpt)])

    return prop


_ND = 16384


_HALF = 5120
_NPA = _HALF + 128


def _make_prop_np(e_pad):
    rpt = _NPA // _NTILE
    nsup = e_pad // (_NTILE * _KCH * _NBUF)
    mesh = plsc.VectorSubcoreMesh(core_axis_name="c", subcore_axis_name="s")
    scratch = (
        [pltpu.VMEM((_KCH,), jnp.int32) for _ in range(2 * _NBUF)]
        + [pltpu.VMEM((_KCH, H), jnp.float32) for _ in range(_NBUF)]
        + [pltpu.VMEM_SHARED((_NPA, H), jnp.float32)]
        + [pltpu.SemaphoreType.DMA for _ in range(3 * _NBUF)]
    )

    @functools.partial(
        pl.kernel,
        mesh=mesh,
        out_type=jax.ShapeDtypeStruct((_NSC, _NPA, H), jnp.float32),
        scratch_types=scratch,
    )
    def prop(feat, srcp, dstp, zrows, out, *refs):
        src_v = refs[0:_NBUF]
        dst_v = refs[_NBUF:2 * _NBUF]
        rows_v = refs[2 * _NBUF:3 * _NBUF]
        acc = refs[3 * _NBUF]
        sem_i = refs[3 * _NBUF + 1: 4 * _NBUF + 1]
        sem_g = refs[4 * _NBUF + 1: 5 * _NBUF + 1]
        sem_s = refs[5 * _NBUF + 1: 6 * _NBUF + 1]
        c = lax.axis_index("c")
        s = lax.axis_index("s")
        pltpu.sync_copy(zrows, acc.at[pl.ds(s * rpt, rpt)])
        plsc.subcore_barrier()

        def idx_start(j, b):
            off = c * e_pad + (j * _NTILE + s) * _KCH
            pltpu.async_copy(srcp.at[pl.ds(off, _KCH)], src_v[b], sem_i[b])
            pltpu.async_copy(dstp.at[pl.ds(off, _KCH)], dst_v[b], sem_i[b])

        for b in range(_NBUF):
            idx_start(b, b)

        def body(g, carry):
            for b in range(_NBUF):
                pltpu.make_async_copy(
                    srcp.at[pl.ds(0, _KCH)], src_v[b], sem_i[b]).wait()
                pltpu.make_async_copy(
                    dstp.at[pl.ds(0, _KCH)], dst_v[b], sem_i[b]).wait()
                pltpu.async_copy(feat.at[src_v[b]], rows_v[b], sem_g[b])
            for b in range(_NBUF):
                pltpu.make_async_copy(
                    feat.at[src_v[b]], rows_v[b], sem_g[b]).wait()
                pltpu.async_copy(rows_v[b], acc.at[dst_v[b]], sem_s[b],
                                 add=True)
            for b in range(_NBUF):
                pltpu.make_async_copy(
                    rows_v[b], acc.at[dst_v[b]], sem_s[b]).wait()

                @pl.when(g + 1 < nsup)
                def _():
                    idx_start((g + 1) * _NBUF + b, b)
            return carry

        lax.fori_loop(0, nsup, body, 0)
        plsc.subcore_barrier()
        pltpu.sync_copy(acc.at[pl.ds(s * rpt, rpt)],
                        out.at[c, pl.ds(s * rpt, rpt)])

    return prop


def _make_deg(e_pad):
    ept = e_pad // (_NSC * _NTILE)
    iters = ept // _KCH
    nsup = iters // _NBUF
    rpt = _ND // _NTILE
    mesh = plsc.VectorSubcoreMesh(core_axis_name="c", subcore_axis_name="s")
    scratch = (
        [pltpu.VMEM((_KCH,), jnp.int32) for _ in range(_NBUF)]
        + [pltpu.VMEM((_KCH,), jnp.float32) for _ in range(_NBUF)]
        + [pltpu.VMEM_SHARED((_ND,), jnp.float32)]
        + [pltpu.SemaphoreType.DMA for _ in range(2 * _NBUF)]
    )

    @functools.partial(
        pl.kernel,
        mesh=mesh,
        out_type=jax.ShapeDtypeStruct((_NSC * _ND,), jnp.float32),
        scratch_types=scratch,
    )
    def deg(vals, dstp, zrow, out, *refs):
        dst_v = refs[0:_NBUF]
        val_v = refs[_NBUF:2 * _NBUF]
        acc = refs[2 * _NBUF]
        sem_i = refs[2 * _NBUF + 1: 3 * _NBUF + 1]
        sem_s = refs[3 * _NBUF + 1: 4 * _NBUF + 1]
        c = lax.axis_index("c")
        s = lax.axis_index("s")
        pltpu.sync_copy(zrow, acc.at[pl.ds(s * rpt, rpt)])
        plsc.subcore_barrier()
        base = (c * _NTILE + s) * ept

        def idx_start(chunk, b):
            off = base + chunk * _KCH
            pltpu.async_copy(dstp.at[pl.ds(off, _KCH)], dst_v[b], sem_i[b])
            pltpu.async_copy(vals.at[pl.ds(off, _KCH)], val_v[b], sem_i[b])

        for b in range(_NBUF):
            idx_start(b, b)

        def body(g, carry):
            for b in range(_NBUF):
                pltpu.make_async_copy(
                    dstp.at[pl.ds(0, _KCH)], dst_v[b], sem_i[b]).wait()
                pltpu.make_async_copy(
                    vals.at[pl.ds(0, _KCH)], val_v[b], sem_i[b]).wait()
                pltpu.async_copy(val_v[b], acc.at[dst_v[b]], sem_s[b],
                                 add=True)
            for b in range(_NBUF):
                pltpu.make_async_copy(
                    val_v[b], acc.at[dst_v[b]], sem_s[b]).wait()

                @pl.when(g + 1 < nsup)
                def _():
                    idx_start((g + 1) * _NBUF + b, b)
            return carry

        lax.fori_loop(0, nsup, body, 0)
        plsc.subcore_barrier()
        pltpu.sync_copy(acc.at[pl.ds(s * rpt, rpt)],
                        out.at[pl.ds(c * _ND + s * rpt, rpt)])

    return deg


_PROP_CACHE = {}
_DEG_CACHE = {}


def _split_edges(srcp, dstp):
    lo = dstp < _HALF
    spread_src = N + srcp % 96
    src2 = jnp.concatenate([jnp.where(lo, srcp, spread_src),
                            jnp.where(lo, spread_src, srcp)])
    dst2 = jnp.concatenate(
        [jnp.where(lo, dstp, _HALF + dstp % 96),
         jnp.where(lo, (N - _HALF) + dstp % 96, dstp - _HALF)])
    return src2, dst2


def _prop_np(feat_pad, src2, dst2):
    if "np" not in _PROP_CACHE:
        _PROP_CACHE["np"] = _make_prop_np(_EPAD_MAX)
    zrows = jnp.zeros((_NPA // _NTILE, H), jnp.float32)
    parts = _PROP_CACHE["np"](feat_pad, src2, dst2, zrows)
    return jnp.concatenate([parts[0, :_HALF], parts[1, :N - _HALF]], axis=0)


def _prop_seg(zp, siota, sbatch):
    key = (G, siota.shape[0])
    if key not in _PROP_CACHE:
        _PROP_CACHE[key] = _make_prop(G, siota.shape[0])
    zrows = jnp.zeros((G // _NTILE, H), jnp.float32)
    parts = _PROP_CACHE[key](zp, siota, sbatch, zrows)
    return parts[0] + parts[1]


def _deg_sc(vals_pad, dstp):
    e_pad = dstp.shape[0]
    if e_pad not in _DEG_CACHE:
        _DEG_CACHE[e_pad] = _make_deg(e_pad)
    zrow = jnp.zeros((_ND // _NTILE,), jnp.float32)
    flat = _DEG_CACHE[e_pad](vals_pad, dstp, zrow)
    return flat[:_ND] + flat[_ND:]


def _pad_edges(src, dst):
    e = src.shape[0]
    fill = N + jnp.arange(_EPAD_MAX - e, dtype=jnp.int32) % 96
    return (jnp.concatenate([src, fill]), jnp.concatenate([dst, fill]))


def _pad_vals(vals, e_pad):
    return jnp.concatenate(
        [vals, jnp.zeros((e_pad - vals.shape[0],), jnp.float32)])


def _scale_pad(u, dinv):
    return jnp.zeros((NP, H), jnp.float32).at[:N].set(u * dinv[:, None])


def _encode_pair(u1, dinv, src2, dst2, b1, W2, b2):
    v1 = _scale_pad(u1, dinv)
    agg1 = _prop_np(v1, src2, dst2) * dinv[:, None]
    h = jnp.maximum(agg1 + b1[None, :], 0.0)
    u2 = mm_bias(h, W2, jnp.zeros((H,), jnp.float32))
    v2 = _scale_pad(u2, dinv)
    z = _prop_np(v2, src2, dst2) * dinv[:, None] + b2[None, :]
    return jnp.zeros((NP, H), jnp.float32).at[:N].set(z)


def kernel(x, edge_index, batch, W1, b1, W2, b2):
    src = edge_index[0]
    dst = edge_index[1]

    akey = jax.random.key(42)
    ka, kb, kw = jax.random.split(akey, 3)
    fmask = jax.random.bernoulli(ka, 0.8, (1, D)).astype(x.dtype)
    x1 = x * fmask
    ew2 = jax.random.bernoulli(kb, 0.8, (E,)).astype(x.dtype)

    u_a = mm_bias(x, W1, jnp.zeros((H,), jnp.float32))
    w1m = W1 * fmask[0][:, None]
    u_b = mm_bias(x, w1m, jnp.zeros((H,), jnp.float32))

    srcp1, dstp1 = _pad_edges(src, dst)
    keep = ew2 > 0.5
    srcp2, dstp2 = _pad_edges(
        jnp.where(keep, src, N), jnp.where(keep, dst, N))

    deg1 = jnp.maximum(
        _deg_sc(_pad_vals(jnp.ones((E,), jnp.float32), _EPAD_MAX), dstp1)[:N], 1.0)
    dinv1 = lax.rsqrt(deg1)
    deg2 = jnp.maximum(_deg_sc(_pad_vals(ew2, _EPAD_MAX), dstp1)[:N], 1.0)
    dinv2 = lax.rsqrt(deg2)

    seg_epad = _EALIGN
    siota = jnp.concatenate([jnp.arange(N, dtype=jnp.int32),
                             jnp.full((seg_epad - N,), N, jnp.int32)])
    sbatch = jnp.concatenate([batch.astype(jnp.int32),
                              jnp.zeros((seg_epad - N,), jnp.int32)])

    def seg_sc(zp):
        return _prop_seg(zp, siota, sbatch)

    s2a, d2a = _split_edges(srcp1, dstp1)
    s2b, d2b = _split_edges(srcp2, dstp2)
    zp_ = _encode_pair(u_a, dinv1, s2a, d2a, b1, W2, b2)
    z, g = zp_[:N], seg_sc(zp_)
    zp1 = _encode_pair(u_b, dinv1, s2a, d2a, b1, W2, b2)
    z1, g1 = zp1[:N], seg_sc(zp1)
    zp2 = _encode_pair(u_a, dinv2, s2b, d2b, b1, W2, b2)
    z2, g2 = zp2[:N], seg_sc(zp2)

    order = jnp.argsort(src)
    src_s = src[order]
    dst_s = dst[order]
    rowptr = jnp.searchsorted(src_s, jnp.arange(N + 1, dtype=jnp.int32)).astype(jnp.int32)

    def walk(key, batch_size, length):
        k0 = jax.random.fold_in(key, 10000)
        cur = jax.random.randint(k0, (batch_size,), 0, N, dtype=jnp.int32)
        es, ed = [], []
        for i in range(length):
            ki = jax.random.fold_in(key, i)
            left = rowptr[cur]
            degc = rowptr[cur + 1] - left
            r = jax.random.randint(ki, (batch_size,), 0, 1 << 30, dtype=jnp.int32)
            idx = jnp.clip(left + r % jnp.maximum(degc, 1), 0, E - 1)
            nxt = jnp.where(degc > 0, dst_s[idx], cur)
            es.append(cur)
            ed.append(nxt)
            cur = nxt
        return jnp.concatenate(es), jnp.concatenate(ed)

    def rw_encode(s, d):
        sp, dp = _pad_edges(s, d)
        degw = jnp.maximum(
            _deg_sc(_pad_vals(jnp.ones(s.shape, jnp.float32), _EPAD_MAX), dp)[:N],
            1.0)
        dinvw = lax.rsqrt(degw)
        sw, dw = _split_edges(sp, dp)
        zwp = _encode_pair(u_a, dinvw, sw, dw, b1, W2, b2)
        return seg_sc(zwp)

    gs3, gs4 = [], []
    for num in range(3):
        k3 = jax.random.fold_in(kw, 2 * num)
        k4 = jax.random.fold_in(kw, 2 * num + 1)
        s3, d3 = walk(k3, 1000, 7 + num)
        s4, d4 = walk(k4, 999, 12 + num)
        gs3.append(rw_encode(s3, d3))
        gs4.append(rw_encode(s4, d4))

    return (z, g, z1, z2, g1, g2, x1, x, tuple(gs3), tuple(gs4))
